# Initial kernel scaffold; baseline (speedup 1.0000x reference)
import jax
import jax.numpy as jnp
from jax.experimental import pallas as pl


def kernel(logits):
    raise NotImplementedError("write your pallas kernel here")


# speedup vs baseline: 19.4889x; 19.4889x over previous
import functools

import jax
import jax.numpy as jnp
from jax import lax
from jax.experimental import pallas as pl

TOPK = 50
TOPP = 0.9
NEG = float("-inf")
GMAX = 64
ROWS_B = 8


def _select_kernel(x_ref, s_ref, *, nrows, ncols):
    x = x_ref[0]
    row_iota = lax.broadcasted_iota(jnp.int32, (TOPK, ncols), 0)

    def cond1(st):
        _, _, _, r, done = st
        return jnp.logical_and(r < TOPK, jnp.logical_not(done))

    def body1(st):
        a, vals, cnts, r, done = st
        cm = jnp.max(a, axis=0, keepdims=True)
        lane_valid = cm > NEG
        eq = jnp.logical_and(a == cm, lane_valid)
        cnt = jnp.sum(eq.astype(jnp.float32), axis=0, keepdims=True)
        cmv = jnp.where(lane_valid, cm, NEG)
        vals = jnp.where(row_iota == r, cmv, vals)
        cnts = jnp.where(row_iota == r, cnt, cnts)
        a = jnp.where(eq, NEG, a)
        m_next = jnp.max(a)
        above = jnp.sum(jnp.where(vals > m_next, cnts, 0.0))
        done = above >= TOPK
        return a, vals, cnts, r + 1, done

    vals0 = jnp.full((TOPK, ncols), NEG, dtype=jnp.float32)
    cnts0 = jnp.zeros((TOPK, ncols), dtype=jnp.float32)
    _, vals, cnts, _, _ = lax.while_loop(
        cond1, body1, (x, vals0, cnts0, jnp.int32(0), jnp.bool_(False))
    )

    giota = lax.broadcasted_iota(jnp.int32, (1, GMAX), 1)

    def cond2(st):
        _, _, _, j, tot = st
        return jnp.logical_and(tot < TOPK, j < GMAX)

    def body2(st):
        v, gv, gc, j, tot = st
        m = jnp.max(v)
        c = jnp.sum(jnp.where(v == m, cnts, 0.0))
        gv = jnp.where(giota == j, m, gv)
        gc = jnp.where(giota == j, c, gc)
        v = jnp.where(v == m, NEG, v)
        return v, gv, gc, j + 1, tot + c

    gv0 = jnp.full((1, GMAX), NEG, dtype=jnp.float32)
    gc0 = jnp.zeros((1, GMAX), dtype=jnp.float32)
    _, gv, gc, _, _ = lax.while_loop(
        cond2, body2, (vals, gv0, gc0, jnp.int32(0), jnp.float32(0.0))
    )

    gvalid = gc > 0.0
    m_top = jnp.max(gv)
    w = jnp.where(gvalid, jnp.exp(gv - m_top), 0.0)
    mass = gc * w
    s_total = jnp.sum(mass)
    tri = (
        lax.broadcasted_iota(jnp.int32, (GMAX, GMAX), 0)
        <= lax.broadcasted_iota(jnp.int32, (GMAX, GMAX), 1)
    ).astype(jnp.float32)
    cum = jnp.dot(mass, tri, preferred_element_type=jnp.float32)
    prev = cum - mass
    thr = TOPP * s_total
    nk = jnp.floor((thr - prev) / w) + 1.0
    nk = jnp.where(w > 0.0, nk, jnp.where(prev <= thr, gc, 0.0))
    nk = jnp.where(gvalid, jnp.clip(nk, 0.0, gc), 0.0)
    kept = nk >= 1.0
    t_p = jnp.min(jnp.where(kept, gv, jnp.inf))
    n_at = jnp.sum(jnp.where(jnp.logical_and(kept, gv == t_p), nk, 0.0))
    c_at = jnp.sum(jnp.where(gv == t_p, gc, 0.0))
    lse = m_top + jnp.log(jnp.sum(nk * w))

    def icut_split():
        eq = x == t_p
        eqf = eq.astype(jnp.float32)
        tri_c = (
            lax.broadcasted_iota(jnp.int32, (ncols, ncols), 0)
            <= lax.broadcasted_iota(jnp.int32, (ncols, ncols), 1)
        ).astype(jnp.float32)
        incol = jnp.dot(eqf, tri_c, preferred_element_type=jnp.float32)
        rowtot = jnp.sum(eqf, axis=1, keepdims=True)
        tri_r = (
            lax.broadcasted_iota(jnp.int32, (nrows, nrows), 0)
            < lax.broadcasted_iota(jnp.int32, (nrows, nrows), 1)
        ).astype(jnp.float32)
        rowprev = jnp.dot(
            rowtot.reshape(1, nrows), tri_r, preferred_element_type=jnp.float32
        ).reshape(nrows, 1)
        pc = incol + rowprev
        hit = jnp.logical_and(eq, pc == n_at)
        flat = lax.broadcasted_iota(jnp.int32, (nrows, ncols), 0) * ncols + (
            lax.broadcasted_iota(jnp.int32, (nrows, ncols), 1)
        )
        return jnp.max(jnp.where(hit, flat, -1))

    icut = lax.cond(n_at >= c_at, lambda: jnp.int32(2**30), icut_split)

    li = lax.broadcasted_iota(jnp.int32, (1, 1, 128), 2)
    out = jnp.where(
        li == 0,
        t_p,
        jnp.where(li == 1, lse, jnp.where(li == 2, icut.astype(jnp.float32), 0.0)),
    )
    s_ref[...] = out


def _apply_kernel(x_ref, s_ref, o_ref):
    x = x_ref[...]
    s = s_ref[...]
    t_p = s[:, 0, 0:1]
    lse = s[:, 0, 1:2]
    icut = s[:, 0, 2:3].astype(jnp.int32)
    vi = lax.broadcasted_iota(jnp.int32, x.shape, 1)
    keep = jnp.logical_or(
        x > t_p, jnp.logical_and(x == t_p, vi <= icut)
    )
    o_ref[...] = jnp.where(keep, x - lse, NEG)


@jax.jit
def kernel(logits):
    b, h, v = logits.shape
    n = b * h
    vp = ((v + 127) // 128) * 128
    nrows = vp // 128
    x2 = logits.reshape(n, v)
    xp = jnp.pad(x2, ((0, 0), (0, vp - v)), constant_values=NEG)
    xp = xp.reshape(n, nrows, 128)

    scal = pl.pallas_call(
        functools.partial(_select_kernel, nrows=nrows, ncols=128),
        grid=(n,),
        in_specs=[pl.BlockSpec((1, nrows, 128), lambda i: (i, 0, 0))],
        out_specs=pl.BlockSpec((1, 1, 128), lambda i: (i, 0, 0)),
        out_shape=jax.ShapeDtypeStruct((n, 1, 128), jnp.float32),
    )(xp)

    rb = ROWS_B if n % ROWS_B == 0 else 1
    out = pl.pallas_call(
        _apply_kernel,
        grid=(n // rb,),
        in_specs=[
            pl.BlockSpec((rb, v), lambda i: (i, 0)),
            pl.BlockSpec((rb, 1, 128), lambda i: (i, 0, 0)),
        ],
        out_specs=pl.BlockSpec((rb, v), lambda i: (i, 0)),
        out_shape=jax.ShapeDtypeStruct((n, v), jnp.float32),
    )(x2, scal)
    return out.reshape(b, h, v)


# speedup vs baseline: 76.2195x; 3.9109x over previous
import functools

import jax
import jax.numpy as jnp
from jax import lax
from jax.experimental import pallas as pl
from jax.experimental.pallas import tpu as pltpu

TOPK = 50
TOPP = 0.9
NEG = float("-inf")
GMAX = 64
RSEL = 8
RAPP = 8


def _select_kernel(x_ref, s_ref, a_ref, *, nrows, ncols, nr):
    x = x_ref[...]
    a_ref[...] = x
    row_iota = lax.broadcasted_iota(jnp.int32, (nr, TOPK, ncols), 1)

    def cond1(st):
        _, _, r, done = st
        return jnp.logical_and(r < TOPK, jnp.sum(done) < nr)

    def body1(st):
        vals, cnts, r, done = st
        a = a_ref[...]
        cm = jnp.max(a, axis=1, keepdims=True)
        lane_valid = cm > NEG
        eq = jnp.logical_and(a == cm, lane_valid)
        cnt = jnp.sum(eq.astype(jnp.float32), axis=1, keepdims=True)
        cmv = jnp.where(lane_valid, cm, NEG)
        vals = jnp.where(row_iota == r, cmv, vals)
        cnts = jnp.where(row_iota == r, cnt, cnts)
        a = jnp.where(eq, NEG, a)
        a_ref[...] = a
        m_next = jnp.max(a, axis=(1, 2), keepdims=True)
        above = jnp.sum(
            jnp.where(vals > m_next, cnts, 0.0), axis=(1, 2), keepdims=True
        )
        done = (above >= TOPK).astype(jnp.float32)
        return vals, cnts, r + 1, done

    vals0 = jnp.full((nr, TOPK, ncols), NEG, dtype=jnp.float32)
    cnts0 = jnp.zeros((nr, TOPK, ncols), dtype=jnp.float32)
    done0 = jnp.zeros((nr, 1, 1), dtype=jnp.float32)
    vals, cnts, _, _ = lax.while_loop(
        cond1, body1, (vals0, cnts0, jnp.int32(0), done0)
    )

    giota = lax.broadcasted_iota(jnp.int32, (nr, 1, GMAX), 2)

    def cond2(st):
        _, _, _, j, tot = st
        return jnp.logical_and(jnp.any(tot < TOPK), j < GMAX)

    def body2(st):
        v, gv, gc, j, tot = st
        active = tot < TOPK
        m = jnp.max(v, axis=(1, 2), keepdims=True)
        c = jnp.sum(
            jnp.where(v == m, cnts, 0.0), axis=(1, 2), keepdims=True
        )
        rec = jnp.logical_and(giota == j, active)
        gv = jnp.where(rec, m, gv)
        gc = jnp.where(rec, c, gc)
        v = jnp.where(jnp.logical_and(v == m, active), NEG, v)
        return v, gv, gc, j + 1, tot + jnp.where(active, c, 0.0)

    gv0 = jnp.full((nr, 1, GMAX), NEG, dtype=jnp.float32)
    gc0 = jnp.zeros((nr, 1, GMAX), dtype=jnp.float32)
    tot0 = jnp.zeros((nr, 1, 1), dtype=jnp.float32)
    _, gv, gc, _, _ = lax.while_loop(
        cond2, body2, (vals, gv0, gc0, jnp.int32(0), tot0)
    )

    gvalid = gc > 0.0
    m_top = jnp.max(gv, axis=2, keepdims=True)
    w = jnp.where(gvalid, jnp.exp(gv - m_top), 0.0)
    mass = gc * w
    s_total = jnp.sum(mass, axis=2, keepdims=True)
    tri = (
        lax.broadcasted_iota(jnp.int32, (GMAX, GMAX), 0)
        <= lax.broadcasted_iota(jnp.int32, (GMAX, GMAX), 1)
    ).astype(jnp.float32)
    cum = jnp.dot(
        mass.reshape(nr, GMAX), tri, preferred_element_type=jnp.float32
    ).reshape(nr, 1, GMAX)
    prev = cum - mass
    thr = TOPP * s_total
    nk = jnp.floor((thr - prev) / w) + 1.0
    nk = jnp.where(w > 0.0, nk, jnp.where(prev <= thr, gc, 0.0))
    nk = jnp.where(gvalid, jnp.clip(nk, 0.0, gc), 0.0)
    kept = nk >= 1.0
    t_p = jnp.min(
        jnp.where(kept, gv, jnp.inf), axis=2, keepdims=True
    )
    n_at = jnp.sum(
        jnp.where(jnp.logical_and(kept, gv == t_p), nk, 0.0),
        axis=2,
        keepdims=True,
    )
    c_at = jnp.sum(jnp.where(gv == t_p, gc, 0.0), axis=2, keepdims=True)
    lse = m_top + jnp.log(jnp.sum(nk * w, axis=2, keepdims=True))

    split = n_at < c_at

    def icut_split():
        eq = x == t_p
        eqf = eq.astype(jnp.float32)
        tri_c = (
            lax.broadcasted_iota(jnp.int32, (ncols, ncols), 0)
            <= lax.broadcasted_iota(jnp.int32, (ncols, ncols), 1)
        ).astype(jnp.float32)
        incol = jnp.stack(
            [
                jnp.dot(eqf[i], tri_c, preferred_element_type=jnp.float32)
                for i in range(nr)
            ],
            axis=0,
        )
        rowtot = jnp.sum(eqf, axis=2)
        tri_r = (
            lax.broadcasted_iota(jnp.int32, (nrows, nrows), 0)
            < lax.broadcasted_iota(jnp.int32, (nrows, nrows), 1)
        ).astype(jnp.float32)
        rowprev = jnp.dot(
            rowtot, tri_r, preferred_element_type=jnp.float32
        ).reshape(nr, nrows, 1)
        pc = incol + rowprev
        hit = jnp.logical_and(eq, pc == n_at)
        flat = lax.broadcasted_iota(
            jnp.int32, (nr, nrows, ncols), 1
        ) * ncols + lax.broadcasted_iota(jnp.int32, (nr, nrows, ncols), 2)
        icr = jnp.max(
            jnp.where(hit, flat, -1), axis=(1, 2), keepdims=True
        )
        return jnp.where(split, icr, 2**30)

    icut = lax.cond(
        jnp.any(split), icut_split, lambda: jnp.full((nr, 1, 1), 2**30, jnp.int32)
    )

    li = lax.broadcasted_iota(jnp.int32, (nr, 1, 128), 2)
    out = jnp.where(
        li == 0,
        t_p,
        jnp.where(li == 1, lse, jnp.where(li == 2, icut.astype(jnp.float32), 0.0)),
    )
    s_ref[...] = out


def _apply_kernel(x_ref, s_ref, o_ref):
    x = x_ref[...]
    s = s_ref[...]
    t_p = s[:, 0, 0:1]
    lse = s[:, 0, 1:2]
    icut = s[:, 0, 2:3].astype(jnp.int32)
    vi = lax.broadcasted_iota(jnp.int32, x.shape, 1)
    keep = jnp.logical_or(x > t_p, jnp.logical_and(x == t_p, vi <= icut))
    o_ref[...] = jnp.where(keep, x - lse, NEG)


@jax.jit
def kernel(logits):
    b, h, v = logits.shape
    n = b * h
    vp = ((v + 127) // 128) * 128
    nrows = vp // 128
    x2 = logits.reshape(n, v)
    xp = jnp.pad(x2, ((0, 0), (0, vp - v)), constant_values=NEG)
    xp = xp.reshape(n, nrows, 128)

    rs = RSEL if n % RSEL == 0 else 1
    scal = pl.pallas_call(
        functools.partial(_select_kernel, nrows=nrows, ncols=128, nr=rs),
        grid=(n // rs,),
        in_specs=[pl.BlockSpec((rs, nrows, 128), lambda i: (i, 0, 0))],
        out_specs=pl.BlockSpec((rs, 1, 128), lambda i: (i, 0, 0)),
        out_shape=jax.ShapeDtypeStruct((n, 1, 128), jnp.float32),
        scratch_shapes=[pltpu.VMEM((rs, nrows, 128), jnp.float32)],
    )(xp)

    rb = RAPP if n % RAPP == 0 else 1
    out = pl.pallas_call(
        _apply_kernel,
        grid=(n // rb,),
        in_specs=[
            pl.BlockSpec((rb, v), lambda i: (i, 0)),
            pl.BlockSpec((rb, 1, 128), lambda i: (i, 0, 0)),
        ],
        out_specs=pl.BlockSpec((rb, v), lambda i: (i, 0)),
        out_shape=jax.ShapeDtypeStruct((n, v), jnp.float32),
    )(x2, scal)
    return out.reshape(b, h, v)


# speedup vs baseline: 129.5826x; 1.7001x over previous
import functools

import jax
import jax.numpy as jnp
from jax import lax
from jax.experimental import pallas as pl
from jax.experimental.pallas import tpu as pltpu

TOPK = 50
TOPP = 0.9
NEG = float("-inf")
GMAX = 64
RB = 8
RFAST = 8


def _group_extract(vals, cnts, nr, nround):
    giota = lax.broadcasted_iota(jnp.int32, (nr, 1, GMAX), 2)

    def cond2(st):
        _, _, _, j, tot = st
        return jnp.logical_and(jnp.any(tot < TOPK), j < GMAX)

    def body2(st):
        v, gv, gc, j, tot = st
        active = tot < TOPK
        m = jnp.max(v, axis=(1, 2), keepdims=True)
        c = jnp.sum(jnp.where(v == m, cnts, 0.0), axis=(1, 2), keepdims=True)
        rec = jnp.logical_and(giota == j, active)
        gv = jnp.where(rec, m, gv)
        gc = jnp.where(rec, c, gc)
        v = jnp.where(jnp.logical_and(v == m, active), NEG, v)
        return v, gv, gc, j + 1, tot + jnp.where(active, c, 0.0)

    gv0 = jnp.full((nr, 1, GMAX), NEG, dtype=jnp.float32)
    gc0 = jnp.zeros((nr, 1, GMAX), dtype=jnp.float32)
    tot0 = jnp.zeros((nr, 1, 1), dtype=jnp.float32)
    _, gv, gc, _, _ = lax.while_loop(
        cond2, body2, (vals, gv0, gc0, jnp.int32(0), tot0)
    )
    return gv, gc


def _fused_kernel(x_ref, o_ref, a_ref, *, nr, v, nrows, ncols):
    x = x_ref[...]
    vmain = (v // ncols) * ncols
    ntail = v - vmain
    xa = x[:, :vmain].reshape(nr, vmain // ncols, ncols)
    if ntail:
        tail = jnp.concatenate(
            [x[:, vmain:], jnp.full((nr, ncols - ntail), NEG, jnp.float32)],
            axis=1,
        ).reshape(nr, 1, ncols)
        xa = jnp.concatenate([xa, tail], axis=1)
    a_ref[...] = xa
    row_iota = lax.broadcasted_iota(jnp.int32, (nr, TOPK, ncols), 1)

    def cond1(st):
        _, _, _, r, done = st
        return jnp.logical_and(r < TOPK, jnp.sum(done) < nr)

    def body1(st):
        vals, cnts, cm, r, done = st
        a = a_ref[...]
        lane_valid = cm > NEG
        eq = jnp.logical_and(a == cm, lane_valid)
        cnt = jnp.sum(eq.astype(jnp.float32), axis=1, keepdims=True)
        cmv = jnp.where(lane_valid, cm, NEG)
        vals = jnp.where(row_iota == r, cmv, vals)
        cnts = jnp.where(row_iota == r, cnt, cnts)
        a = jnp.where(eq, NEG, a)
        a_ref[...] = a
        cm = jnp.max(a, axis=1, keepdims=True)
        m_next = jnp.max(cm, axis=2, keepdims=True)
        above = jnp.sum(
            jnp.where(vals > m_next, cnts, 0.0), axis=(1, 2), keepdims=True
        )
        done = (above >= TOPK).astype(jnp.float32)
        return vals, cnts, cm, r + 1, done

    vals0 = jnp.full((nr, TOPK, ncols), NEG, dtype=jnp.float32)
    cnts0 = jnp.zeros((nr, TOPK, ncols), dtype=jnp.float32)
    cm0 = jnp.max(xa, axis=1, keepdims=True)
    done0 = jnp.zeros((nr, 1, 1), dtype=jnp.float32)
    vals, cnts, _, rstop, _ = lax.while_loop(
        cond1, body1, (vals0, cnts0, cm0, jnp.int32(0), done0)
    )

    gv, gc = lax.cond(
        rstop <= RFAST,
        lambda: _group_extract(
            vals[:, :RFAST, :], cnts[:, :RFAST, :], nr, RFAST
        ),
        lambda: _group_extract(vals, cnts, nr, TOPK),
    )

    gvalid = gc > 0.0
    m_top = jnp.max(gv, axis=2, keepdims=True)
    w = jnp.where(gvalid, jnp.exp(gv - m_top), 0.0)
    mass = gc * w
    s_total = jnp.sum(mass, axis=2, keepdims=True)
    tri = (
        lax.broadcasted_iota(jnp.int32, (GMAX, GMAX), 0)
        <= lax.broadcasted_iota(jnp.int32, (GMAX, GMAX), 1)
    ).astype(jnp.float32)
    cum = jnp.dot(
        mass.reshape(nr, GMAX), tri, preferred_element_type=jnp.float32
    ).reshape(nr, 1, GMAX)
    prev = cum - mass
    thr = TOPP * s_total
    nk = jnp.floor((thr - prev) / w) + 1.0
    nk = jnp.where(w > 0.0, nk, jnp.where(prev <= thr, gc, 0.0))
    nk = jnp.where(gvalid, jnp.clip(nk, 0.0, gc), 0.0)
    kept = nk >= 1.0
    t_p = jnp.min(jnp.where(kept, gv, jnp.inf), axis=2, keepdims=True)
    n_at = jnp.sum(
        jnp.where(jnp.logical_and(kept, gv == t_p), nk, 0.0),
        axis=2,
        keepdims=True,
    )
    c_at = jnp.sum(jnp.where(gv == t_p, gc, 0.0), axis=2, keepdims=True)
    lse = m_top + jnp.log(jnp.sum(nk * w, axis=2, keepdims=True))

    split = n_at < c_at

    def icut_split():
        xb = xa
        eq = xb == t_p
        eqf = eq.astype(jnp.float32)
        tri_c = (
            lax.broadcasted_iota(jnp.int32, (ncols, ncols), 0)
            <= lax.broadcasted_iota(jnp.int32, (ncols, ncols), 1)
        ).astype(jnp.float32)
        incol = jnp.stack(
            [
                jnp.dot(eqf[i], tri_c, preferred_element_type=jnp.float32)
                for i in range(nr)
            ],
            axis=0,
        )
        rowtot = jnp.sum(eqf, axis=2)
        tri_r = (
            lax.broadcasted_iota(jnp.int32, (nrows, nrows), 0)
            < lax.broadcasted_iota(jnp.int32, (nrows, nrows), 1)
        ).astype(jnp.float32)
        rowprev = jnp.dot(
            rowtot, tri_r, preferred_element_type=jnp.float32
        ).reshape(nr, nrows, 1)
        pc = incol + rowprev
        hit = jnp.logical_and(eq, pc == n_at)
        flat = lax.broadcasted_iota(
            jnp.int32, (nr, nrows, ncols), 1
        ) * ncols + lax.broadcasted_iota(jnp.int32, (nr, nrows, ncols), 2)
        icr = jnp.max(jnp.where(hit, flat, -1), axis=(1, 2), keepdims=True)
        return jnp.where(split, icr, 2**30)

    icut = lax.cond(
        jnp.any(split),
        icut_split,
        lambda: jnp.full((nr, 1, 1), 2**30, jnp.int32),
    )

    tp2 = t_p.reshape(nr, 1)
    lse2 = lse.reshape(nr, 1)
    ic2 = icut.reshape(nr, 1)
    vi = lax.broadcasted_iota(jnp.int32, (nr, v), 1)
    keep = jnp.logical_or(
        x > tp2, jnp.logical_and(x == tp2, vi <= ic2)
    )
    o_ref[...] = jnp.where(keep, x - lse2, NEG)


@jax.jit
def kernel(logits):
    b, h, v = logits.shape
    n = b * h
    nrows = (v + 127) // 128
    x2 = logits.reshape(n, v)
    rb = RB if n % RB == 0 else 1
    out = pl.pallas_call(
        functools.partial(_fused_kernel, nr=rb, v=v, nrows=nrows, ncols=128),
        grid=(n // rb,),
        in_specs=[pl.BlockSpec((rb, v), lambda i: (i, 0))],
        out_specs=pl.BlockSpec((rb, v), lambda i: (i, 0)),
        out_shape=jax.ShapeDtypeStruct((n, v), jnp.float32),
        scratch_shapes=[pltpu.VMEM((rb, nrows, 128), jnp.float32)],
    )(x2)
    return out.reshape(b, h, v)


# speedup vs baseline: 138.4097x; 1.0681x over previous
import functools

import jax
import jax.numpy as jnp
from jax import lax
from jax.experimental import pallas as pl
from jax.experimental.pallas import tpu as pltpu

TOPK = 50
TOPP = 0.9
NEG = float("-inf")
GMAX = 64
RB = 8
NSUB = 4
RFAST = 3


def _group_extract(vals, cnts, nr, nsub):
    nb = nr * nsub
    giota = lax.broadcasted_iota(jnp.int32, (nr, 1, GMAX), 2)
    zero_sub = jnp.zeros((nr, nsub), jnp.float32)

    def body2(j, st):
        v, gv, gc, tot = st
        active = tot < TOPK
        mm = jnp.max(v, axis=(1, 2), keepdims=True)
        m_row = jnp.max(
            mm.reshape(nr, nsub), axis=1, keepdims=True
        ).reshape(nr, 1, 1)
        m_b = (m_row.reshape(nr, 1) + zero_sub).reshape(nb, 1, 1)
        cw = jnp.sum(
            jnp.where(v == m_b, cnts, 0.0), axis=(1, 2), keepdims=True
        )
        c_row = jnp.sum(
            cw.reshape(nr, nsub), axis=1, keepdims=True
        ).reshape(nr, 1, 1)
        rec = jnp.logical_and(giota == j, active)
        gv = jnp.where(rec, m_row, gv)
        gc = jnp.where(rec, c_row, gc)
        act_b = (
            active.astype(jnp.float32).reshape(nr, 1) + zero_sub
        ).reshape(nb, 1, 1) > 0.0
        v = jnp.where(jnp.logical_and(v == m_b, act_b), NEG, v)
        return v, gv, gc, tot + jnp.where(active, c_row, 0.0)

    gv0 = jnp.full((nr, 1, GMAX), NEG, dtype=jnp.float32)
    gc0 = jnp.zeros((nr, 1, GMAX), dtype=jnp.float32)
    tot0 = jnp.zeros((nr, 1, 1), dtype=jnp.float32)
    _, gv, gc, _ = lax.fori_loop(
        0, TOPK, body2, (vals, gv0, gc0, tot0), unroll=2
    )
    return gv, gc


def _fused_kernel(x_ref, o_ref, a_ref, *, nr, v, nrows, ncols):
    x = x_ref[...]
    nb = nr * NSUB
    sub = nrows // NSUB
    vmain = (v // ncols) * ncols
    ntail = v - vmain
    parts = [x[:, :vmain].reshape(nr, vmain // ncols, ncols)]
    nfill = nrows - vmain // ncols
    if ntail:
        parts.append(
            jnp.concatenate(
                [x[:, vmain:], jnp.full((nr, ncols - ntail), NEG, jnp.float32)],
                axis=1,
            ).reshape(nr, 1, ncols)
        )
        nfill -= 1
    if nfill:
        parts.append(jnp.full((nr, nfill, ncols), NEG, jnp.float32))
    xa = jnp.concatenate(parts, axis=1)
    ab = xa.reshape(nb, sub, ncols)
    a_ref[...] = ab
    row_iota = lax.broadcasted_iota(jnp.int32, (nb, TOPK, ncols), 1)
    zero_sub = jnp.zeros((nr, NSUB), jnp.float32)

    def cond1(st):
        _, _, _, r, done = st
        return jnp.logical_and(r < TOPK, jnp.sum(done) < nr)

    def body1(st):
        vals, cnts, cm, r, done = st
        a = a_ref[...]
        eq = a == cm
        cnt = jnp.sum(eq.astype(jnp.float32), axis=1, keepdims=True)
        vals = jnp.where(row_iota == r, cm, vals)
        cnts = jnp.where(row_iota == r, cnt, cnts)
        a = jnp.where(eq, NEG, a)
        a_ref[...] = a
        cm = jnp.max(a, axis=1, keepdims=True)
        m_row = jnp.max(
            jnp.max(cm, axis=2).reshape(nr, NSUB), axis=1, keepdims=True
        ).reshape(nr, 1, 1)
        m_b = (m_row.reshape(nr, 1) + zero_sub).reshape(nb, 1, 1)
        above = jnp.sum(
            jnp.where(vals > m_b, cnts, 0.0), axis=(1, 2), keepdims=True
        )
        above_row = jnp.sum(above.reshape(nr, NSUB), axis=1, keepdims=True)
        done = (above_row >= TOPK).astype(jnp.float32)
        return vals, cnts, cm, r + 1, done

    vals0 = jnp.full((nb, TOPK, ncols), NEG, dtype=jnp.float32)
    cnts0 = jnp.zeros((nb, TOPK, ncols), dtype=jnp.float32)
    cm0 = jnp.max(ab, axis=1, keepdims=True)
    done0 = jnp.zeros((nr, 1), dtype=jnp.float32)
    vals, cnts, _, rstop, _ = lax.while_loop(
        cond1, body1, (vals0, cnts0, cm0, jnp.int32(0), done0)
    )

    gv, gc = lax.cond(
        rstop <= RFAST,
        lambda: _group_extract(
            vals[:, :RFAST, :], cnts[:, :RFAST, :], nr, NSUB
        ),
        lambda: _group_extract(vals, cnts, nr, NSUB),
    )

    gvalid = gc > 0.0
    m_top = jnp.max(gv, axis=2, keepdims=True)
    w = jnp.where(gvalid, jnp.exp(gv - m_top), 0.0)
    mass = gc * w
    s_total = jnp.sum(mass, axis=2, keepdims=True)
    tri = (
        lax.broadcasted_iota(jnp.int32, (GMAX, GMAX), 0)
        <= lax.broadcasted_iota(jnp.int32, (GMAX, GMAX), 1)
    ).astype(jnp.float32)
    cum = jnp.dot(
        mass.reshape(nr, GMAX), tri, preferred_element_type=jnp.float32
    ).reshape(nr, 1, GMAX)
    prev = cum - mass
    thr = TOPP * s_total
    nk = jnp.floor((thr - prev) / w) + 1.0
    nk = jnp.where(w > 0.0, nk, jnp.where(prev <= thr, gc, 0.0))
    nk = jnp.where(gvalid, jnp.clip(nk, 0.0, gc), 0.0)
    kept = nk >= 1.0
    t_p = jnp.min(jnp.where(kept, gv, jnp.inf), axis=2, keepdims=True)
    n_at = jnp.sum(
        jnp.where(jnp.logical_and(kept, gv == t_p), nk, 0.0),
        axis=2,
        keepdims=True,
    )
    c_at = jnp.sum(jnp.where(gv == t_p, gc, 0.0), axis=2, keepdims=True)
    lse = m_top + jnp.log(jnp.sum(nk * w, axis=2, keepdims=True))

    split = n_at < c_at

    def icut_split():
        xb = xa
        eq = xb == t_p
        eqf = eq.astype(jnp.float32)
        tri_c = (
            lax.broadcasted_iota(jnp.int32, (ncols, ncols), 0)
            <= lax.broadcasted_iota(jnp.int32, (ncols, ncols), 1)
        ).astype(jnp.float32)
        incol = jnp.stack(
            [
                jnp.dot(eqf[i], tri_c, preferred_element_type=jnp.float32)
                for i in range(nr)
            ],
            axis=0,
        )
        rowtot = jnp.sum(eqf, axis=2)
        tri_r = (
            lax.broadcasted_iota(jnp.int32, (nrows, nrows), 0)
            < lax.broadcasted_iota(jnp.int32, (nrows, nrows), 1)
        ).astype(jnp.float32)
        rowprev = jnp.dot(
            rowtot, tri_r, preferred_element_type=jnp.float32
        ).reshape(nr, nrows, 1)
        pc = incol + rowprev
        hit = jnp.logical_and(eq, pc == n_at)
        flat = lax.broadcasted_iota(
            jnp.int32, (nr, nrows, ncols), 1
        ) * ncols + lax.broadcasted_iota(jnp.int32, (nr, nrows, ncols), 2)
        icr = jnp.max(jnp.where(hit, flat, -1), axis=(1, 2), keepdims=True)
        return jnp.where(split, icr, 2**30)

    icut = lax.cond(
        jnp.any(split),
        icut_split,
        lambda: jnp.full((nr, 1, 1), 2**30, jnp.int32),
    )

    tp2 = t_p.reshape(nr, 1)
    lse2 = lse.reshape(nr, 1)
    ic2 = icut.reshape(nr, 1)
    vi = lax.broadcasted_iota(jnp.int32, (nr, v), 1)
    keep = jnp.logical_or(x > tp2, jnp.logical_and(x == tp2, vi <= ic2))
    o_ref[...] = jnp.where(keep, x - lse2, NEG)


@jax.jit
def kernel(logits):
    b, h, v = logits.shape
    n = b * h
    nrows = (v + 127) // 128
    nrows = ((nrows + 8 * NSUB - 1) // (8 * NSUB)) * (8 * NSUB)
    x2 = logits.reshape(n, v)
    rb = RB if n % RB == 0 else 1
    out = pl.pallas_call(
        functools.partial(_fused_kernel, nr=rb, v=v, nrows=nrows, ncols=128),
        grid=(n // rb,),
        in_specs=[pl.BlockSpec((rb, v), lambda i: (i, 0))],
        out_specs=pl.BlockSpec((rb, v), lambda i: (i, 0)),
        out_shape=jax.ShapeDtypeStruct((n, v), jnp.float32),
        scratch_shapes=[
            pltpu.VMEM((rb * NSUB, nrows // NSUB, 128), jnp.float32)
        ],
    )(x2)
    return out.reshape(b, h, v)


# speedup vs baseline: 149.0099x; 1.0766x over previous
import functools

import jax
import jax.numpy as jnp
from jax import lax
from jax.experimental import pallas as pl
from jax.experimental.pallas import tpu as pltpu

TOPK = 50
TOPP = 0.9
NEG = float("-inf")
GMAX = 64
RB = 8
NSUB = 4
RFAST = 3


def _group_extract(vals, cnts, nr):
    giota = lax.broadcasted_iota(jnp.int32, (nr, 1, GMAX), 2)

    def body2(j, st):
        v, gv, gc, tot = st
        active = tot < TOPK
        m = jnp.max(v, axis=(1, 2), keepdims=True)
        c = jnp.sum(jnp.where(v == m, cnts, 0.0), axis=(1, 2), keepdims=True)
        rec = jnp.logical_and(giota == j, active)
        gv = jnp.where(rec, m, gv)
        gc = jnp.where(rec, c, gc)
        v = jnp.where(jnp.logical_and(v == m, active), NEG, v)
        return v, gv, gc, tot + jnp.where(active, c, 0.0)

    gv0 = jnp.full((nr, 1, GMAX), NEG, dtype=jnp.float32)
    gc0 = jnp.zeros((nr, 1, GMAX), dtype=jnp.float32)
    tot0 = jnp.zeros((nr, 1, 1), dtype=jnp.float32)
    _, gv, gc, _ = lax.fori_loop(
        0, TOPK, body2, (vals, gv0, gc0, tot0), unroll=4
    )
    return gv, gc


def _fused_kernel(x_ref, o_ref, a_ref, *, nr, v, nrows, ncols):
    x = x_ref[...]
    nb = nr * NSUB
    sub = nrows // NSUB
    vmain = (v // ncols) * ncols
    ntail = v - vmain
    parts = [x[:, :vmain].reshape(nr, vmain // ncols, ncols)]
    nfill = nrows - vmain // ncols
    if ntail:
        parts.append(
            jnp.concatenate(
                [x[:, vmain:], jnp.full((nr, ncols - ntail), NEG, jnp.float32)],
                axis=1,
            ).reshape(nr, 1, ncols)
        )
        nfill -= 1
    if nfill:
        parts.append(jnp.full((nr, nfill, ncols), NEG, jnp.float32))
    xa = jnp.concatenate(parts, axis=1)
    ab = xa.reshape(nb, sub, ncols)
    a_ref[...] = ab
    row_iota = lax.broadcasted_iota(jnp.int32, (nb, TOPK, ncols), 1)
    zero_sub = jnp.zeros((nr, NSUB), jnp.float32)

    def cond1(st):
        _, _, _, r, done = st
        return jnp.logical_and(r < TOPK, jnp.sum(done) < nr)

    def body1(st):
        vals, cnts, cm, r, done = st
        a = a_ref[...]
        eq = a == cm
        cnt = jnp.sum(eq.astype(jnp.float32), axis=1, keepdims=True)
        vals = jnp.where(row_iota == r, cm, vals)
        cnts = jnp.where(row_iota == r, cnt, cnts)
        a = jnp.where(eq, NEG, a)
        a_ref[...] = a
        cm = jnp.max(a, axis=1, keepdims=True)
        m_row = jnp.max(
            jnp.max(cm, axis=2).reshape(nr, NSUB), axis=1, keepdims=True
        ).reshape(nr, 1, 1)
        m_b = (m_row.reshape(nr, 1) + zero_sub).reshape(nb, 1, 1)
        above = jnp.sum(
            jnp.where(vals > m_b, cnts, 0.0), axis=(1, 2), keepdims=True
        )
        above_row = jnp.sum(above.reshape(nr, NSUB), axis=1, keepdims=True)
        done = (above_row >= TOPK).astype(jnp.float32)
        return vals, cnts, cm, r + 1, done

    vals0 = jnp.full((nb, TOPK, ncols), NEG, dtype=jnp.float32)
    cnts0 = jnp.zeros((nb, TOPK, ncols), dtype=jnp.float32)
    cm0 = jnp.max(ab, axis=1, keepdims=True)
    done0 = jnp.zeros((nr, 1), dtype=jnp.float32)
    vals, cnts, _, rstop, _ = lax.while_loop(
        cond1, body1, (vals0, cnts0, cm0, jnp.int32(0), done0)
    )

    gv, gc = lax.cond(
        rstop <= RFAST,
        lambda: _group_extract(
            vals[:, :RFAST, :].reshape(nr, NSUB * RFAST, 128),
            cnts[:, :RFAST, :].reshape(nr, NSUB * RFAST, 128),
            nr,
        ),
        lambda: _group_extract(
            vals.reshape(nr, NSUB * TOPK, 128),
            cnts.reshape(nr, NSUB * TOPK, 128),
            nr,
        ),
    )

    gvalid = gc > 0.0
    m_top = jnp.max(gv, axis=2, keepdims=True)
    w = jnp.where(gvalid, jnp.exp(gv - m_top), 0.0)
    mass = gc * w
    s_total = jnp.sum(mass, axis=2, keepdims=True)
    tri = (
        lax.broadcasted_iota(jnp.int32, (GMAX, GMAX), 0)
        <= lax.broadcasted_iota(jnp.int32, (GMAX, GMAX), 1)
    ).astype(jnp.float32)
    cum = jnp.dot(
        mass.reshape(nr, GMAX), tri, preferred_element_type=jnp.float32
    ).reshape(nr, 1, GMAX)
    prev = cum - mass
    thr = TOPP * s_total
    nk = jnp.floor((thr - prev) / w) + 1.0
    nk = jnp.where(w > 0.0, nk, jnp.where(prev <= thr, gc, 0.0))
    nk = jnp.where(gvalid, jnp.clip(nk, 0.0, gc), 0.0)
    kept = nk >= 1.0
    t_p = jnp.min(jnp.where(kept, gv, jnp.inf), axis=2, keepdims=True)
    n_at = jnp.sum(
        jnp.where(jnp.logical_and(kept, gv == t_p), nk, 0.0),
        axis=2,
        keepdims=True,
    )
    c_at = jnp.sum(jnp.where(gv == t_p, gc, 0.0), axis=2, keepdims=True)
    lse = m_top + jnp.log(jnp.sum(nk * w, axis=2, keepdims=True))

    split = n_at < c_at

    def icut_split():
        xb = xa
        eq = xb == t_p
        eqf = eq.astype(jnp.float32)
        tri_c = (
            lax.broadcasted_iota(jnp.int32, (ncols, ncols), 0)
            <= lax.broadcasted_iota(jnp.int32, (ncols, ncols), 1)
        ).astype(jnp.float32)
        incol = jnp.stack(
            [
                jnp.dot(eqf[i], tri_c, preferred_element_type=jnp.float32)
                for i in range(nr)
            ],
            axis=0,
        )
        rowtot = jnp.sum(eqf, axis=2)
        tri_r = (
            lax.broadcasted_iota(jnp.int32, (nrows, nrows), 0)
            < lax.broadcasted_iota(jnp.int32, (nrows, nrows), 1)
        ).astype(jnp.float32)
        rowprev = jnp.dot(
            rowtot, tri_r, preferred_element_type=jnp.float32
        ).reshape(nr, nrows, 1)
        pc = incol + rowprev
        hit = jnp.logical_and(eq, pc == n_at)
        flat = lax.broadcasted_iota(
            jnp.int32, (nr, nrows, ncols), 1
        ) * ncols + lax.broadcasted_iota(jnp.int32, (nr, nrows, ncols), 2)
        icr = jnp.max(jnp.where(hit, flat, -1), axis=(1, 2), keepdims=True)
        return jnp.where(split, icr, 2**30)

    icut = lax.cond(
        jnp.any(split),
        icut_split,
        lambda: jnp.full((nr, 1, 1), 2**30, jnp.int32),
    )

    tp2 = t_p.reshape(nr, 1)
    lse2 = lse.reshape(nr, 1)
    ic2 = icut.reshape(nr, 1)
    vi = lax.broadcasted_iota(jnp.int32, (nr, v), 1)
    keep = jnp.logical_or(x > tp2, jnp.logical_and(x == tp2, vi <= ic2))
    o_ref[...] = jnp.where(keep, x - lse2, NEG)


@jax.jit
def kernel(logits):
    b, h, v = logits.shape
    n = b * h
    nrows = (v + 127) // 128
    nrows = ((nrows + 8 * NSUB - 1) // (8 * NSUB)) * (8 * NSUB)
    x2 = logits.reshape(n, v)
    rb = RB if n % RB == 0 else 1
    out = pl.pallas_call(
        functools.partial(_fused_kernel, nr=rb, v=v, nrows=nrows, ncols=128),
        grid=(n // rb,),
        in_specs=[pl.BlockSpec((rb, v), lambda i: (i, 0))],
        out_specs=pl.BlockSpec((rb, v), lambda i: (i, 0)),
        out_shape=jax.ShapeDtypeStruct((n, v), jnp.float32),
        scratch_shapes=[
            pltpu.VMEM((rb * NSUB, nrows // NSUB, 128), jnp.float32)
        ],
    )(x2)
    return out.reshape(b, h, v)


# speedup vs baseline: 161.8308x; 1.0860x over previous
import functools

import jax
import jax.numpy as jnp
from jax import lax
from jax.experimental import pallas as pl
from jax.experimental.pallas import tpu as pltpu

TOPK = 50
TOPP = 0.9
NEG = float("-inf")
GMAX = 64
RB = 8
NSUB = 4
RFAST = 3
RCAP = 56
RCHK = 8


def _group_extract(vals, cnts, nr):
    giota = lax.broadcasted_iota(jnp.int32, (nr, 1, GMAX), 2)

    def body2(j, st):
        v, gv, gc, tot = st
        active = tot < TOPK
        m = jnp.max(v, axis=(1, 2), keepdims=True)
        c = jnp.sum(jnp.where(v == m, cnts, 0.0), axis=(1, 2), keepdims=True)
        rec = jnp.logical_and(giota == j, active)
        gv = jnp.where(rec, m, gv)
        gc = jnp.where(rec, c, gc)
        v = jnp.where(jnp.logical_and(v == m, active), NEG, v)
        return v, gv, gc, tot + jnp.where(active, c, 0.0)

    gv0 = jnp.full((nr, 1, GMAX), NEG, dtype=jnp.float32)
    gc0 = jnp.zeros((nr, 1, GMAX), dtype=jnp.float32)
    tot0 = jnp.zeros((nr, 1, 1), dtype=jnp.float32)
    _, gv, gc, _ = lax.fori_loop(
        0, TOPK, body2, (vals, gv0, gc0, tot0), unroll=4
    )
    return gv, gc


def _fused_kernel(x_ref, o_ref, a_ref, vals_ref, cnts_ref, *, nr, v, nrows, ncols):
    x = x_ref[...]
    nb = nr * NSUB
    sub = nrows // NSUB
    vmain = (v // ncols) * ncols
    ntail = v - vmain
    parts = [x[:, :vmain].reshape(nr, vmain // ncols, ncols)]
    nfill = nrows - vmain // ncols
    if ntail:
        parts.append(
            jnp.concatenate(
                [x[:, vmain:], jnp.full((nr, ncols - ntail), NEG, jnp.float32)],
                axis=1,
            ).reshape(nr, 1, ncols)
        )
        nfill -= 1
    if nfill:
        parts.append(jnp.full((nr, nfill, ncols), NEG, jnp.float32))
    xa = jnp.concatenate(parts, axis=1)
    ab = xa.reshape(nb, sub, ncols)
    a_ref[...] = ab
    zero_sub = jnp.zeros((nr, NSUB), jnp.float32)
    vals_ref[...] = jnp.full((nb, RCAP, ncols), NEG, jnp.float32)
    cnts_ref[...] = jnp.zeros((nb, RCAP, ncols), jnp.float32)

    def cond1(st):
        _, r, done = st
        return jnp.logical_and(r < TOPK, jnp.sum(done) < nr)

    def body1(st):
        cm, r, done = st
        a = a_ref[...]
        eq = a == cm
        cnt = jnp.sum(eq.astype(jnp.float32), axis=1, keepdims=True)
        vals_ref[:, pl.ds(jnp.minimum(r, RCAP - 1), 1), :] = cm
        cnts_ref[:, pl.ds(jnp.minimum(r, RCAP - 1), 1), :] = cnt
        a = jnp.where(eq, NEG, a)
        a_ref[...] = a
        cm = jnp.max(a, axis=1, keepdims=True)
        m_row = jnp.max(
            jnp.max(cm, axis=2).reshape(nr, NSUB), axis=1, keepdims=True
        ).reshape(nr, 1, 1)
        m_b = (m_row.reshape(nr, 1) + zero_sub).reshape(nb, 1, 1)
        above = jnp.sum(
            jnp.where(vals_ref[:, :RCHK, :] > m_b, cnts_ref[:, :RCHK, :], 0.0),
            axis=(1, 2),
            keepdims=True,
        )
        above_row = jnp.sum(above.reshape(nr, NSUB), axis=1, keepdims=True)
        done = (above_row >= TOPK).astype(jnp.float32)
        return cm, r + 1, done

    cm0 = jnp.max(ab, axis=1, keepdims=True)
    done0 = jnp.zeros((nr, 1), dtype=jnp.float32)
    _, rstop, _ = lax.while_loop(
        cond1, body1, (cm0, jnp.int32(0), done0)
    )

    gv, gc = lax.cond(
        rstop <= RFAST,
        lambda: _group_extract(
            vals_ref[:, :RFAST, :].reshape(nr, NSUB * RFAST, 128),
            cnts_ref[:, :RFAST, :].reshape(nr, NSUB * RFAST, 128),
            nr,
        ),
        lambda: _group_extract(
            vals_ref[...].reshape(nr, NSUB * RCAP, 128),
            cnts_ref[...].reshape(nr, NSUB * RCAP, 128),
            nr,
        ),
    )

    gvalid = gc > 0.0
    m_top = jnp.max(gv, axis=2, keepdims=True)
    w = jnp.where(gvalid, jnp.exp(gv - m_top), 0.0)
    mass = gc * w
    s_total = jnp.sum(mass, axis=2, keepdims=True)
    tri = (
        lax.broadcasted_iota(jnp.int32, (GMAX, GMAX), 0)
        <= lax.broadcasted_iota(jnp.int32, (GMAX, GMAX), 1)
    ).astype(jnp.float32)
    cum = jnp.dot(
        mass.reshape(nr, GMAX), tri, preferred_element_type=jnp.float32
    ).reshape(nr, 1, GMAX)
    prev = cum - mass
    thr = TOPP * s_total
    nk = jnp.floor((thr - prev) / w) + 1.0
    nk = jnp.where(w > 0.0, nk, jnp.where(prev <= thr, gc, 0.0))
    nk = jnp.where(gvalid, jnp.clip(nk, 0.0, gc), 0.0)
    kept = nk >= 1.0
    t_p = jnp.min(jnp.where(kept, gv, jnp.inf), axis=2, keepdims=True)
    n_at = jnp.sum(
        jnp.where(jnp.logical_and(kept, gv == t_p), nk, 0.0),
        axis=2,
        keepdims=True,
    )
    c_at = jnp.sum(jnp.where(gv == t_p, gc, 0.0), axis=2, keepdims=True)
    lse = m_top + jnp.log(jnp.sum(nk * w, axis=2, keepdims=True))

    split = n_at < c_at

    def icut_split():
        xb = xa
        eq = xb == t_p
        eqf = eq.astype(jnp.float32)
        tri_c = (
            lax.broadcasted_iota(jnp.int32, (ncols, ncols), 0)
            <= lax.broadcasted_iota(jnp.int32, (ncols, ncols), 1)
        ).astype(jnp.float32)
        incol = jnp.stack(
            [
                jnp.dot(eqf[i], tri_c, preferred_element_type=jnp.float32)
                for i in range(nr)
            ],
            axis=0,
        )
        rowtot = jnp.sum(eqf, axis=2)
        tri_r = (
            lax.broadcasted_iota(jnp.int32, (nrows, nrows), 0)
            < lax.broadcasted_iota(jnp.int32, (nrows, nrows), 1)
        ).astype(jnp.float32)
        rowprev = jnp.dot(
            rowtot, tri_r, preferred_element_type=jnp.float32
        ).reshape(nr, nrows, 1)
        pc = incol + rowprev
        hit = jnp.logical_and(eq, pc == n_at)
        flat = lax.broadcasted_iota(
            jnp.int32, (nr, nrows, ncols), 1
        ) * ncols + lax.broadcasted_iota(jnp.int32, (nr, nrows, ncols), 2)
        icr = jnp.max(jnp.where(hit, flat, -1), axis=(1, 2), keepdims=True)
        return jnp.where(split, icr, 2**30)

    icut = lax.cond(
        jnp.any(split),
        icut_split,
        lambda: jnp.full((nr, 1, 1), 2**30, jnp.int32),
    )

    tp2 = t_p.reshape(nr, 1)
    lse2 = lse.reshape(nr, 1)
    ic2 = icut.reshape(nr, 1)
    vi = lax.broadcasted_iota(jnp.int32, (nr, v), 1)
    keep = jnp.logical_or(x > tp2, jnp.logical_and(x == tp2, vi <= ic2))
    o_ref[...] = jnp.where(keep, x - lse2, NEG)


@jax.jit
def kernel(logits):
    b, h, v = logits.shape
    n = b * h
    nrows = (v + 127) // 128
    nrows = ((nrows + 8 * NSUB - 1) // (8 * NSUB)) * (8 * NSUB)
    x2 = logits.reshape(n, v)
    rb = RB if n % RB == 0 else 1
    out = pl.pallas_call(
        functools.partial(_fused_kernel, nr=rb, v=v, nrows=nrows, ncols=128),
        grid=(n // rb,),
        in_specs=[pl.BlockSpec((rb, v), lambda i: (i, 0))],
        out_specs=pl.BlockSpec((rb, v), lambda i: (i, 0)),
        out_shape=jax.ShapeDtypeStruct((n, v), jnp.float32),
        scratch_shapes=[
            pltpu.VMEM((rb * NSUB, nrows // NSUB, 128), jnp.float32),
            pltpu.VMEM((rb * NSUB, RCAP, 128), jnp.float32),
            pltpu.VMEM((rb * NSUB, RCAP, 128), jnp.float32),
        ],
    )(x2)
    return out.reshape(b, h, v)
